# Initial kernel scaffold; baseline (speedup 1.0000x reference)
import jax
import jax.numpy as jnp
from jax.experimental import pallas as pl


def kernel(x, edge_index, W1, b1, g1, be1, a1, W2, b2, g2, be2, a2, W3, b3, g3, be3, a3, W4, b4, g4, be4, a4, Wl, bl):
    raise NotImplementedError("write your pallas kernel here")


# speedup vs baseline: 9.9885x; 9.9885x over previous
import functools

import jax
import jax.numpy as jnp
from jax import lax
from jax.experimental import pallas as pl
from jax.experimental.pallas import tpu as pltpu
from jax.experimental.pallas import tpu_sc as plsc

N = 10000
E = 320000
D = 128
OUT = 64
NC = 2
NS = 16
NW = NC * NS
C = 128
NCH = -(-E // (NW * C))
EP = NW * C * NCH
NPAD = 10112
ZR = NPAD // NS
EPS = 1e-5

_mesh = plsc.VectorSubcoreMesh(core_axis_name="c", subcore_axis_name="s",
                               num_cores=NC, num_subcores=NS)


@functools.partial(
    pl.kernel,
    out_type=jax.ShapeDtypeStruct((NC, NPAD, 16), jnp.float32),
    mesh=_mesh,
    scratch_types=[
        pltpu.VMEM((NCH, C), jnp.int32),
        pltpu.VMEM((C, 16), jnp.float32),
        pltpu.VMEM_SHARED((NPAD, 16), jnp.float32),
    ],
)
def _sc_degree(dst3, ones, zrows, out, dst_v, ones_v, acc):
    c = lax.axis_index("c")
    s = lax.axis_index("s")
    w = s * NC + c
    pltpu.sync_copy(zrows.at[pl.ds(s * ZR, ZR)], acc.at[pl.ds(s * ZR, ZR)])
    pltpu.sync_copy(dst3.at[w], dst_v)
    pltpu.sync_copy(ones.at[:], ones_v)
    plsc.subcore_barrier()

    def body(j, carry):
        pltpu.sync_copy(ones_v, acc.at[dst_v.at[j]], add=True)
        return carry

    lax.fori_loop(0, NCH, body, 0)
    plsc.subcore_barrier()
    pltpu.sync_copy(acc.at[pl.ds(s * ZR, ZR)], out.at[c, pl.ds(s * ZR, ZR)])


@functools.partial(
    pl.kernel,
    out_type=jax.ShapeDtypeStruct((NC, NPAD, D), jnp.float32),
    mesh=_mesh,
    scratch_types=[
        pltpu.VMEM((NCH, C), jnp.int32),
        pltpu.VMEM((NCH, C), jnp.int32),
        pltpu.VMEM((C, D), jnp.float32),
        pltpu.VMEM_SHARED((NPAD, D), jnp.float32),
        pltpu.SemaphoreType.DMA,
    ],
)
def _sc_aggregate(hp, src3, dst3, zrows, out, src_v, dst_v, rows_v, acc, sem):
    c = lax.axis_index("c")
    s = lax.axis_index("s")
    w = s * NC + c
    pltpu.sync_copy(zrows.at[pl.ds(s * ZR, ZR)], acc.at[pl.ds(s * ZR, ZR)])
    pltpu.sync_copy(src3.at[w], src_v)
    pltpu.sync_copy(dst3.at[w], dst_v)
    plsc.subcore_barrier()

    def body(j, carry):
        pltpu.async_copy(hp.at[src_v.at[j]], rows_v, sem).wait()
        pltpu.sync_copy(rows_v, acc.at[dst_v.at[j]], add=True)
        return carry

    lax.fori_loop(0, NCH, body, 0)
    plsc.subcore_barrier()
    pltpu.sync_copy(acc.at[pl.ds(s * ZR, ZR)], out.at[c, pl.ds(s * ZR, ZR)])


def _tc_first_body(x_ref, degp_ref, w1_ref, hp_ref, dinv_ref):
    deg = degp_ref[0, :N, 0:1] + degp_ref[1, :N, 0:1] + 1.0
    dinv = lax.rsqrt(deg)
    h = jnp.dot(x_ref[...], w1_ref[...], preferred_element_type=jnp.float32)
    hp_ref[...] = h * dinv
    dinv_ref[...] = dinv


_tc_first = pl.pallas_call(
    _tc_first_body,
    out_shape=[
        jax.ShapeDtypeStruct((N, D), jnp.float32),
        jax.ShapeDtypeStruct((N, 1), jnp.float32),
    ],
)


def _tc_mid_body(p_ref, hp_ref, dinv_ref, b_ref, g_ref, be_ref, a_ref, wn_ref,
                 out_ref):
    dinv = dinv_ref[...]
    y = dinv * (p_ref[0, :N] + p_ref[1, :N] + hp_ref[...]) + b_ref[...]
    mean = jnp.mean(y, axis=0, keepdims=True)
    sub = y - a_ref[...] * mean
    var = jnp.mean(sub * sub, axis=0, keepdims=True)
    z = jnp.maximum(g_ref[...] * sub / jnp.sqrt(var + EPS) + be_ref[...], 0.0)
    out_ref[...] = jnp.dot(z, wn_ref[...],
                           preferred_element_type=jnp.float32) * dinv


_tc_mid = pl.pallas_call(
    _tc_mid_body,
    out_shape=jax.ShapeDtypeStruct((N, D), jnp.float32),
)


def _tc_final_body(p_ref, hp_ref, dinv_ref, b_ref, g_ref, be_ref, a_ref,
                   wl_ref, bl_ref, out_ref):
    dinv = dinv_ref[...]
    y = dinv * (p_ref[0, :N] + p_ref[1, :N] + hp_ref[...]) + b_ref[...]
    mean = jnp.mean(y, axis=0, keepdims=True)
    sub = y - a_ref[...] * mean
    var = jnp.mean(sub * sub, axis=0, keepdims=True)
    z = jnp.maximum(g_ref[...] * sub / jnp.sqrt(var + EPS) + be_ref[...], 0.0)
    pooled = jnp.mean(z, axis=0, keepdims=True)
    out_ref[...] = jnp.dot(pooled, wl_ref[...],
                           preferred_element_type=jnp.float32) + bl_ref[...]


_tc_final = pl.pallas_call(
    _tc_final_body,
    out_shape=jax.ShapeDtypeStruct((1, OUT), jnp.float32),
)


def kernel(x, edge_index, W1, b1, g1, be1, a1, W2, b2, g2, be2, a2, W3, b3,
           g3, be3, a3, W4, b4, g4, be4, a4, Wl, bl):
    pad = EP - E
    src3 = jnp.concatenate(
        [edge_index[0], jnp.zeros((pad,), jnp.int32)]).reshape(NW, NCH, C)
    dst3 = jnp.concatenate(
        [edge_index[1], jnp.full((pad,), N, jnp.int32)]).reshape(NW, NCH, C)
    zrows = jnp.zeros((NPAD, D), jnp.float32)
    z16 = jnp.zeros((NPAD, 16), jnp.float32)
    ones16 = jnp.ones((C, 16), jnp.float32)

    degp = _sc_degree(dst3, ones16, z16)
    hp, dinv = _tc_first(x, degp, W1)

    b1r, b2r, b3r, b4r = (v.reshape(1, D) for v in (b1, b2, b3, b4))
    params = [(b1r, g1.reshape(1, D), be1.reshape(1, D), a1.reshape(1, D)),
              (b2r, g2.reshape(1, D), be2.reshape(1, D), a2.reshape(1, D)),
              (b3r, g3.reshape(1, D), be3.reshape(1, D), a3.reshape(1, D))]
    for (br, gr, ber, ar), Wn in zip(params, (W2, W3, W4)):
        p = _sc_aggregate(hp, src3, dst3, zrows)
        hp = _tc_mid(p, hp, dinv, br, gr, ber, ar, Wn)

    p = _sc_aggregate(hp, src3, dst3, zrows)
    return _tc_final(p, hp, dinv, b4.reshape(1, D), g4.reshape(1, D),
                     be4.reshape(1, D), a4.reshape(1, D), Wl,
                     bl.reshape(1, OUT))


# speedup vs baseline: 10.6141x; 1.0626x over previous
import functools

import jax
import jax.numpy as jnp
from jax import lax
from jax.experimental import pallas as pl
from jax.experimental.pallas import tpu as pltpu
from jax.experimental.pallas import tpu_sc as plsc

N = 10000
E = 320000
D = 128
OUT = 64
NC = 2
NS = 16
NW = NC * NS
C = 128
NCH = 79
HALF = NCH // 2
EP = NW * C * NCH
NPAD = 10112
ZR = NPAD // NS
EPS = 1e-5

_mesh = plsc.VectorSubcoreMesh(core_axis_name="c", subcore_axis_name="s",
                               num_cores=NC, num_subcores=NS)


@functools.partial(
    pl.kernel,
    out_type=jax.ShapeDtypeStruct((NC, NPAD, D), jnp.float32),
    mesh=_mesh,
    scratch_types=[
        pltpu.VMEM((NCH, C), jnp.int32),
        pltpu.VMEM((C, D), jnp.float32),
        pltpu.VMEM_SHARED((NPAD, D), jnp.float32),
    ],
)
def _sc_degree(dst3, ones, zrows, out, dst_v, ones_v, acc):
    c = lax.axis_index("c")
    s = lax.axis_index("s")
    w = s * NC + c
    pltpu.sync_copy(zrows.at[pl.ds(s * ZR, ZR)], acc.at[pl.ds(s * ZR, ZR)])
    pltpu.sync_copy(dst3.at[w], dst_v)
    pltpu.sync_copy(ones.at[:], ones_v)
    plsc.subcore_barrier()

    def body(j, carry):
        pltpu.sync_copy(ones_v, acc.at[dst_v.at[j]], add=True)
        return carry

    lax.fori_loop(0, NCH, body, 0)
    plsc.subcore_barrier()
    pltpu.sync_copy(acc.at[pl.ds(s * ZR, ZR)], out.at[c, pl.ds(s * ZR, ZR)])


@functools.partial(
    pl.kernel,
    out_type=jax.ShapeDtypeStruct((NC, NPAD, D), jnp.float32),
    mesh=_mesh,
    scratch_types=[
        pltpu.VMEM((NCH, C), jnp.int32),
        pltpu.VMEM((NCH, C), jnp.int32),
        pltpu.VMEM((C, D), jnp.float32),
        pltpu.VMEM_SHARED((NPAD, D), jnp.float32),
        pltpu.SemaphoreType.DMA,
    ],
)
def _sc_aggregate(hp, src3, dst3, zrows, out, src_v, dst_v, rows0, acc, sem0):
    c = lax.axis_index("c")
    s = lax.axis_index("s")
    w = s * NC + c
    pltpu.sync_copy(zrows.at[pl.ds(s * ZR, ZR)], acc.at[pl.ds(s * ZR, ZR)])
    pltpu.sync_copy(src3.at[w], src_v)
    pltpu.sync_copy(dst3.at[w], dst_v)
    plsc.subcore_barrier()

    def body(j, carry):
        pltpu.async_copy(hp.at[src_v.at[j]], rows0, sem0).wait()
        pltpu.sync_copy(rows0, acc.at[dst_v.at[j]], add=True)
        return carry

    lax.fori_loop(0, NCH, body, 0)
    plsc.subcore_barrier()
    pltpu.sync_copy(acc.at[pl.ds(s * ZR, ZR)], out.at[c, pl.ds(s * ZR, ZR)])


def _tc_first_body(x_ref, degp_ref, w1_ref, hp_ref, dinv_ref):
    deg = degp_ref[0, :N, 0:1] + degp_ref[1, :N, 0:1] + 1.0
    dinv = lax.rsqrt(deg)
    h = jnp.dot(x_ref[...], w1_ref[...], preferred_element_type=jnp.float32)
    hp_ref[...] = h * dinv
    dinv_ref[...] = dinv


_tc_first = pl.pallas_call(
    _tc_first_body,
    out_shape=[
        jax.ShapeDtypeStruct((N, D), jnp.float32),
        jax.ShapeDtypeStruct((N, 1), jnp.float32),
    ],
)


def _tc_mid_body(p_ref, hp_ref, dinv_ref, b_ref, g_ref, be_ref, a_ref, wn_ref,
                 out_ref):
    dinv = dinv_ref[...]
    y = dinv * (p_ref[0, :N] + p_ref[1, :N] + hp_ref[...]) + b_ref[...]
    mean = jnp.mean(y, axis=0, keepdims=True)
    sub = y - a_ref[...] * mean
    var = jnp.mean(sub * sub, axis=0, keepdims=True)
    z = jnp.maximum(g_ref[...] * sub / jnp.sqrt(var + EPS) + be_ref[...], 0.0)
    out_ref[...] = jnp.dot(z, wn_ref[...],
                           preferred_element_type=jnp.float32) * dinv


_tc_mid = pl.pallas_call(
    _tc_mid_body,
    out_shape=jax.ShapeDtypeStruct((N, D), jnp.float32),
)


def _tc_final_body(p_ref, hp_ref, dinv_ref, b_ref, g_ref, be_ref, a_ref,
                   wl_ref, bl_ref, out_ref):
    dinv = dinv_ref[...]
    y = dinv * (p_ref[0, :N] + p_ref[1, :N] + hp_ref[...]) + b_ref[...]
    mean = jnp.mean(y, axis=0, keepdims=True)
    sub = y - a_ref[...] * mean
    var = jnp.mean(sub * sub, axis=0, keepdims=True)
    z = jnp.maximum(g_ref[...] * sub / jnp.sqrt(var + EPS) + be_ref[...], 0.0)
    pooled = jnp.mean(z, axis=0, keepdims=True)
    out_ref[...] = jnp.dot(pooled, wl_ref[...],
                           preferred_element_type=jnp.float32) + bl_ref[...]


_tc_final = pl.pallas_call(
    _tc_final_body,
    out_shape=jax.ShapeDtypeStruct((1, OUT), jnp.float32),
)


def kernel(x, edge_index, W1, b1, g1, be1, a1, W2, b2, g2, be2, a2, W3, b3,
           g3, be3, a3, W4, b4, g4, be4, a4, Wl, bl):
    pad = EP - E
    src3 = jnp.concatenate(
        [edge_index[0], jnp.zeros((pad,), jnp.int32)]).reshape(NW, NCH, C)
    dst3 = jnp.concatenate(
        [edge_index[1], jnp.full((pad,), N, jnp.int32)]).reshape(NW, NCH, C)
    zrows = jnp.zeros((NPAD, D), jnp.float32)
    onesC = jnp.ones((C, D), jnp.float32)

    degp = _sc_degree(dst3, onesC, zrows)
    hp, dinv = _tc_first(x, degp, W1)

    b1r, b2r, b3r, b4r = (v.reshape(1, D) for v in (b1, b2, b3, b4))
    params = [(b1r, g1.reshape(1, D), be1.reshape(1, D), a1.reshape(1, D)),
              (b2r, g2.reshape(1, D), be2.reshape(1, D), a2.reshape(1, D)),
              (b3r, g3.reshape(1, D), be3.reshape(1, D), a3.reshape(1, D))]
    for (br, gr, ber, ar), Wn in zip(params, (W2, W3, W4)):
        p = _sc_aggregate(hp, src3, dst3, zrows)
        hp = _tc_mid(p, hp, dinv, br, gr, ber, ar, Wn)

    p = _sc_aggregate(hp, src3, dst3, zrows)
    return _tc_final(p, hp, dinv, b4.reshape(1, D), g4.reshape(1, D),
                     be4.reshape(1, D), a4.reshape(1, D), Wl,
                     bl.reshape(1, OUT))
